# Initial kernel scaffold; baseline (speedup 1.0000x reference)
import jax
import jax.numpy as jnp
from jax.experimental import pallas as pl


def kernel(inputs, indices):
    raise NotImplementedError("write your pallas kernel here")


# speedup vs baseline: 1.9635x; 1.9635x over previous
import jax
import jax.numpy as jnp
from jax.experimental import pallas as pl

_N_ABS = 1000
_B = 16384
_BLOCK_ROWS = 512
_GRID = _B // _BLOCK_ROWS


def _onehot_block(idx_ref, out_ref):
    idx = idx_ref[0, 0, :]
    cols = jax.lax.broadcasted_iota(jnp.int32, (_BLOCK_ROWS, _N_ABS), 1)
    out_ref[...] = (cols == idx[:, None]).astype(jnp.float32)


def kernel(inputs, indices):
    del inputs
    idx3 = indices.reshape(_GRID, 1, _BLOCK_ROWS)
    return pl.pallas_call(
        _onehot_block,
        grid=(_GRID,),
        in_specs=[pl.BlockSpec((1, 1, _BLOCK_ROWS), lambda i: (i, 0, 0))],
        out_specs=pl.BlockSpec((_BLOCK_ROWS, _N_ABS), lambda i: (i, 0)),
        out_shape=jax.ShapeDtypeStruct((_B, _N_ABS), jnp.float32),
    )(idx3)
